# baseline (speedup 1.0000x reference)
import functools

import jax
import jax.numpy as jnp
from jax import lax
from jax.experimental import pallas as pl
from jax.experimental.pallas import tpu as pltpu
from jax.experimental.pallas import tpu_sc as plsc

N_NODES = 10000
N_EDGES = 320000
NUM_GRAPHS = 64
F = 128

NC = 2
NS = 16
NW = NC * NS

NP = 10112
BR = NP // 8
ROWS_PER_TILE = NP // NS

K = 32
CH = 320
EPW = CH * K
EP = NW * EPW
NB = 4
CHA = 512
CHB = 128

_HIGH = lax.Precision.HIGHEST


@functools.lru_cache(maxsize=None)
def _sc_kernels():
    mesh = plsc.VectorSubcoreMesh(
        core_axis_name="c", subcore_axis_name="s",
        num_cores=NC, num_subcores=NS)

    @functools.partial(
        pl.kernel,
        out_type=jax.ShapeDtypeStruct((NC, NP, F), jnp.float32),
        mesh=mesh,
        scratch_types=[
            pltpu.VMEM((CH, K), jnp.int32),
            pltpu.VMEM((K, F), jnp.float32),
            pltpu.VMEM_SHARED((NP, F), jnp.float32),
        ],
    )
    def deg_sc(dst_hbm, z16_hbm, ones_hbm, out_hbm, didx, ones_v, degsh):
        c = lax.axis_index("c")
        s = lax.axis_index("s")
        w = s * NC + c
        pltpu.sync_copy(dst_hbm.at[w], didx)
        pltpu.sync_copy(ones_hbm, ones_v)
        pltpu.sync_copy(z16_hbm.at[pl.ds(s * ROWS_PER_TILE, ROWS_PER_TILE)],
                        degsh.at[pl.ds(s * ROWS_PER_TILE, ROWS_PER_TILE)])
        plsc.subcore_barrier()

        def body(ch, tok):
            pltpu.sync_copy(ones_v, degsh.at[didx.at[ch]], add=True)
            return tok

        lax.fori_loop(0, CH, body, 0)
        plsc.subcore_barrier()
        pltpu.sync_copy(degsh.at[pl.ds(s * ROWS_PER_TILE, ROWS_PER_TILE)],
                        out_hbm.at[c, pl.ds(s * ROWS_PER_TILE, ROWS_PER_TILE)])

    @functools.partial(
        pl.kernel,
        out_type=jax.ShapeDtypeStruct((NC, NP, F), jnp.float32),
        mesh=mesh,
        scratch_types=[
            pltpu.VMEM((2 * NB, K), jnp.int32),
            pltpu.VMEM((2 * NB, K), jnp.int32),
            pltpu.VMEM((NB, K, F), jnp.float32),
            pltpu.VMEM_SHARED((NP, F), jnp.float32),
        ] + [pltpu.SemaphoreType.DMA] * (5 * NB),
    )
    def edge_sc(src_hbm, dst_hbm, y_hbm, z_hbm, out_hbm,
                sidx, didx, gbuf, agg, *sems):
        ssems, dsems, gsems = sems[:2 * NB], sems[2 * NB:4 * NB], sems[4 * NB:]
        c = lax.axis_index("c")
        s = lax.axis_index("s")
        w = s * NC + c
        nch = jnp.where(c == 0, CHA, CHB)

        pltpu.sync_copy(z_hbm.at[pl.ds(s * ROWS_PER_TILE, ROWS_PER_TILE)],
                        agg.at[pl.ds(s * ROWS_PER_TILE, ROWS_PER_TILE)])
        plsc.subcore_barrier()

        def stage_i(ch, islot):
            pltpu.async_copy(src_hbm.at[w, ch], sidx.at[islot], ssems[islot])
            pltpu.async_copy(dst_hbm.at[w, ch], didx.at[islot], dsems[islot])

        def stage_g(ch, islot, gslot):
            pltpu.make_async_copy(
                src_hbm.at[w, ch], sidx.at[islot], ssems[islot]).wait()
            pltpu.async_copy(
                y_hbm.at[sidx.at[islot]], gbuf.at[gslot], gsems[gslot])

        def stage_s(ch, islot, gslot):
            pltpu.make_async_copy(
                y_hbm.at[sidx.at[islot]], gbuf.at[gslot],
                gsems[gslot]).wait()
            pltpu.make_async_copy(
                dst_hbm.at[w, ch], didx.at[islot], dsems[islot]).wait()
            pltpu.sync_copy(gbuf.at[gslot], agg.at[didx.at[islot]], add=True)

        NI = 2 * NB

        @pl.when(nch > 0)
        def _():
            for v in range(NI):
                stage_i(v, v)
                if v >= NB:
                    stage_g(v - NB, (v - NB) % NI, (v - NB) % NB)

        def body(g, tok):
            for u in range(NI):
                ch = NI + g * NI + u
                stage_s(ch - NI, u, u % NB)
                stage_i(ch, u)
                stage_g(ch - NB, (u + NB) % NI, u % NB)
            return tok

        lax.fori_loop(0, (nch - NI) // NI, body, 0)

        @pl.when(nch > 0)
        def _():
            for v in range(NI):
                ch = nch + v
                stage_s(ch - NI, v, v % NB)
                if v < NB:
                    stage_g(ch - NB, (v + NI - NB) % NI, v % NB)
        plsc.subcore_barrier()
        pltpu.sync_copy(agg.at[pl.ds(s * ROWS_PER_TILE, ROWS_PER_TILE)],
                        out_hbm.at[c, pl.ds(s * ROWS_PER_TILE, ROWS_PER_TILE)])

    return deg_sc, edge_sc


def _prep_body(dega_ref, degb_ref, x_ref, w_ref, y_ref, dinv_ref):
    deg = dega_ref[:, :1] + degb_ref[:, :1] + 1.0
    dinv = lax.rsqrt(deg)
    xw = jnp.dot(x_ref[...], w_ref[...],
                 preferred_element_type=jnp.float32, precision=_HIGH)
    y_ref[...] = xw * dinv
    dinv_ref[...] = dinv


def _prep_tc(dega, degb, xp, W1):
    return pl.pallas_call(
        _prep_body,
        grid=(8,),
        in_specs=[
            pl.BlockSpec((BR, F), lambda i: (i, 0)),
            pl.BlockSpec((BR, F), lambda i: (i, 0)),
            pl.BlockSpec((BR, F), lambda i: (i, 0)),
            pl.BlockSpec((F, F), lambda i: (0, 0)),
        ],
        out_specs=[
            pl.BlockSpec((BR, F), lambda i: (i, 0)),
            pl.BlockSpec((BR, 1), lambda i: (i, 0)),
        ],
        out_shape=[
            jax.ShapeDtypeStruct((NP, F), jnp.float32),
            jax.ShapeDtypeStruct((NP, 1), jnp.float32),
        ],
    )(dega, degb, xp, W1)


def _mid_body(dinv_ref, agga_ref, aggb_ref, y_ref, b_ref, w_ref, yn_ref):
    dinv = dinv_ref[...]
    h = jnp.maximum(
        dinv * (agga_ref[...] + aggb_ref[...] + y_ref[...]) + b_ref[...], 0.0)
    yn_ref[...] = jnp.dot(h, w_ref[...], preferred_element_type=jnp.float32,
                          precision=_HIGH) * dinv


def _mid_tc(dinv, agga, aggb, y, b, W):
    return pl.pallas_call(
        _mid_body,
        grid=(8,),
        in_specs=[
            pl.BlockSpec((BR, 1), lambda i: (i, 0)),
            pl.BlockSpec((BR, F), lambda i: (i, 0)),
            pl.BlockSpec((BR, F), lambda i: (i, 0)),
            pl.BlockSpec((BR, F), lambda i: (i, 0)),
            pl.BlockSpec((1, F), lambda i: (0, 0)),
            pl.BlockSpec((F, F), lambda i: (0, 0)),
        ],
        out_specs=pl.BlockSpec((BR, F), lambda i: (i, 0)),
        out_shape=jax.ShapeDtypeStruct((NP, F), jnp.float32),
    )(dinv, agga, aggb, y, b, W)


def _final_body(dinv_ref, agga_ref, aggb_ref, y_ref, b_ref, batch_ref,
                lw1_ref, lb1_ref, lw2_ref, lb2_ref, ow_ref, ob_ref,
                out_ref, pooled_scr, cnt_scr):
    i = pl.program_id(0)
    dinv = dinv_ref[...]
    h = jnp.maximum(
        dinv * (agga_ref[...] + aggb_ref[...] + y_ref[...]) + b_ref[...], 0.0)
    brow = jnp.broadcast_to(batch_ref[0], (NUM_GRAPHS, BR))
    giota = lax.broadcasted_iota(jnp.int32, (NUM_GRAPHS, BR), 0)
    oh = (brow == giota).astype(jnp.float32)
    pooled = jnp.dot(oh, h, preferred_element_type=jnp.float32,
                     precision=_HIGH)
    cnt = jnp.sum(oh, axis=1, keepdims=True)

    @pl.when(i == 0)
    def _():
        pooled_scr[...] = pooled
        cnt_scr[...] = cnt

    @pl.when(i > 0)
    def _():
        pooled_scr[...] += pooled
        cnt_scr[...] += cnt

    @pl.when(i == 7)
    def _():
        g = pooled_scr[...] / jnp.maximum(cnt_scr[...], 1.0)
        g = jnp.maximum(
            jnp.dot(g, lw1_ref[...], preferred_element_type=jnp.float32,
                    precision=_HIGH) + lb1_ref[...], 0.0)
        g = jnp.maximum(
            jnp.dot(g, lw2_ref[...], preferred_element_type=jnp.float32,
                    precision=_HIGH) + lb2_ref[...], 0.0)
        out_ref[...] = jnp.dot(g, ow_ref[...],
                               preferred_element_type=jnp.float32,
                               precision=_HIGH) + ob_ref[...]


def _final_tc(dinv, agga, aggb, y, b, batchp, LW1, Lb1, LW2, Lb2, OW, Ob):
    full = lambda shape: pl.BlockSpec(shape, lambda i: tuple(0 for _ in shape))
    return pl.pallas_call(
        _final_body,
        grid=(8,),
        in_specs=[
            pl.BlockSpec((BR, 1), lambda i: (i, 0)),
            pl.BlockSpec((BR, F), lambda i: (i, 0)),
            pl.BlockSpec((BR, F), lambda i: (i, 0)),
            pl.BlockSpec((BR, F), lambda i: (i, 0)),
            full((1, F)),
            pl.BlockSpec((1, 1, BR), lambda i: (i, 0, 0)),
            full((F, F)),
            full((1, F)),
            full((F, 64)),
            full((1, 64)),
            full((64, 10)),
            full((1, 10)),
        ],
        out_specs=pl.BlockSpec((NUM_GRAPHS, 10), lambda i: (0, 0)),
        out_shape=jax.ShapeDtypeStruct((NUM_GRAPHS, 10), jnp.float32),
        scratch_shapes=[
            pltpu.VMEM((NUM_GRAPHS, F), jnp.float32),
            pltpu.VMEM((NUM_GRAPHS, 1), jnp.float32),
        ],
    )(dinv, agga, aggb, y, b, batchp, LW1, Lb1, LW2, Lb2, OW, Ob)


def kernel(x, edge_index, batch, W1, b1, W2, b2, W3, b3,
           LW1, Lb1, LW2, Lb2, OW, Ob):
    src = edge_index[0].astype(jnp.int32)
    dst = edge_index[1].astype(jnp.int32)
    pad_e = EP - N_EDGES
    src_flat = jnp.concatenate([src, jnp.zeros((pad_e,), jnp.int32)])
    dst_flat = jnp.concatenate(
        [dst, N_NODES + jnp.arange(pad_e, dtype=jnp.int32)
         % (NP - N_NODES)])

    def asym(flat):
        ca = 16 * CHA * K
        a0 = flat[:ca].reshape(16, CHA, K)
        a1 = jnp.pad(flat[ca:].reshape(16, CHB, K),
                     ((0, 0), (0, CHA - CHB), (0, 0)))
        return jnp.stack([a0, a1], axis=1).reshape(NW, CHA, K)

    srcp = asym(src_flat)
    dstp = asym(dst_flat)
    dst_deg = dst_flat.reshape(NW, CH, K)
    xp = jnp.zeros((NP, F), jnp.float32).at[:N_NODES].set(x)
    batchp = jnp.concatenate(
        [batch.astype(jnp.int32),
         jnp.full((NP - N_NODES,), 127, jnp.int32)]).reshape(8, 1, BR)
    zeros_f = jnp.zeros((NP, F), jnp.float32)
    ones_f = jnp.ones((K, F), jnp.float32)

    deg_sc, edge_sc = _sc_kernels()
    deg = deg_sc(dst_deg, zeros_f, ones_f)
    y1, dinv = _prep_tc(deg[0], deg[1], xp, W1)
    agg1 = edge_sc(srcp, dstp, y1, zeros_f)
    y2 = _mid_tc(dinv, agg1[0], agg1[1], y1, b1.reshape(1, F), W2)
    agg2 = edge_sc(srcp, dstp, y2, zeros_f)
    y3 = _mid_tc(dinv, agg2[0], agg2[1], y2, b2.reshape(1, F), W3)
    agg3 = edge_sc(srcp, dstp, y3, zeros_f)
    return _final_tc(dinv, agg3[0], agg3[1], y3, b3.reshape(1, F), batchp,
                     LW1, Lb1.reshape(1, F), LW2, Lb2.reshape(1, 64),
                     OW, Ob.reshape(1, 10))


# speedup vs baseline: 1.1848x; 1.1848x over previous
import functools

import jax
import jax.numpy as jnp
from jax import lax
from jax.experimental import pallas as pl
from jax.experimental.pallas import tpu as pltpu
from jax.experimental.pallas import tpu_sc as plsc

N_NODES = 10000
N_EDGES = 320000
NUM_GRAPHS = 64
F = 128

NC = 2
NS = 16
NW = NC * NS

NP = 10112
BR = NP // 8
ROWS_PER_TILE = NP // NS

K = 32
CH = 320
EPW = CH * K
EP = NW * EPW
NB = 4
CHA = 576
CHB = 64

_HIGH = lax.Precision.HIGHEST


@functools.lru_cache(maxsize=None)
def _sc_kernels():
    mesh = plsc.VectorSubcoreMesh(
        core_axis_name="c", subcore_axis_name="s",
        num_cores=NC, num_subcores=NS)

    @functools.partial(
        pl.kernel,
        out_type=jax.ShapeDtypeStruct((NC, NP, F), jnp.float32),
        mesh=mesh,
        scratch_types=[
            pltpu.VMEM((CH, K), jnp.int32),
            pltpu.VMEM((K, F), jnp.float32),
            pltpu.VMEM_SHARED((NP, F), jnp.float32),
        ],
    )
    def deg_sc(dst_hbm, z16_hbm, ones_hbm, out_hbm, didx, ones_v, degsh):
        c = lax.axis_index("c")
        s = lax.axis_index("s")
        w = s * NC + c
        pltpu.sync_copy(dst_hbm.at[w], didx)
        pltpu.sync_copy(ones_hbm, ones_v)
        pltpu.sync_copy(z16_hbm.at[pl.ds(s * ROWS_PER_TILE, ROWS_PER_TILE)],
                        degsh.at[pl.ds(s * ROWS_PER_TILE, ROWS_PER_TILE)])
        plsc.subcore_barrier()

        def body(ch, tok):
            pltpu.sync_copy(ones_v, degsh.at[didx.at[ch]], add=True)
            return tok

        lax.fori_loop(0, CH, body, 0)
        plsc.subcore_barrier()
        pltpu.sync_copy(degsh.at[pl.ds(s * ROWS_PER_TILE, ROWS_PER_TILE)],
                        out_hbm.at[c, pl.ds(s * ROWS_PER_TILE, ROWS_PER_TILE)])

    @functools.partial(
        pl.kernel,
        out_type=jax.ShapeDtypeStruct((NC, NP, F), jnp.float32),
        mesh=mesh,
        scratch_types=[
            pltpu.VMEM((2 * NB, K), jnp.int32),
            pltpu.VMEM((2 * NB, K), jnp.int32),
            pltpu.VMEM((NB, K, F), jnp.float32),
            pltpu.VMEM_SHARED((NP, F), jnp.float32),
        ] + [pltpu.SemaphoreType.DMA] * (5 * NB),
    )
    def edge_sc(src_hbm, dst_hbm, y_hbm, z_hbm, out_hbm,
                sidx, didx, gbuf, agg, *sems):
        ssems, dsems, gsems = sems[:2 * NB], sems[2 * NB:4 * NB], sems[4 * NB:]
        c = lax.axis_index("c")
        s = lax.axis_index("s")
        w = s * NC + c
        nch = jnp.where(c == 0, CHA, CHB)

        pltpu.sync_copy(z_hbm.at[pl.ds(s * ROWS_PER_TILE, ROWS_PER_TILE)],
                        agg.at[pl.ds(s * ROWS_PER_TILE, ROWS_PER_TILE)])
        plsc.subcore_barrier()

        def stage_i(ch, islot):
            pltpu.async_copy(src_hbm.at[w, ch], sidx.at[islot], ssems[islot])
            pltpu.async_copy(dst_hbm.at[w, ch], didx.at[islot], dsems[islot])

        def stage_g(ch, islot, gslot):
            pltpu.make_async_copy(
                src_hbm.at[w, ch], sidx.at[islot], ssems[islot]).wait()
            pltpu.async_copy(
                y_hbm.at[sidx.at[islot]], gbuf.at[gslot], gsems[gslot])

        def stage_s(ch, islot, gslot):
            pltpu.make_async_copy(
                y_hbm.at[sidx.at[islot]], gbuf.at[gslot],
                gsems[gslot]).wait()
            pltpu.make_async_copy(
                dst_hbm.at[w, ch], didx.at[islot], dsems[islot]).wait()
            pltpu.sync_copy(gbuf.at[gslot], agg.at[didx.at[islot]], add=True)

        NI = 2 * NB

        @pl.when(nch > 0)
        def _():
            for v in range(NI):
                stage_i(v, v)
                if v >= NB:
                    stage_g(v - NB, (v - NB) % NI, (v - NB) % NB)

        def body(g, tok):
            for u in range(NI):
                ch = NI + g * NI + u
                stage_s(ch - NI, u, u % NB)
                stage_i(ch, u)
                stage_g(ch - NB, (u + NB) % NI, u % NB)
            return tok

        lax.fori_loop(0, (nch - NI) // NI, body, 0)

        @pl.when(nch > 0)
        def _():
            for v in range(NI):
                ch = nch + v
                stage_s(ch - NI, v, v % NB)
                if v < NB:
                    stage_g(ch - NB, (v + NI - NB) % NI, v % NB)
        plsc.subcore_barrier()
        pltpu.sync_copy(agg.at[pl.ds(s * ROWS_PER_TILE, ROWS_PER_TILE)],
                        out_hbm.at[c, pl.ds(s * ROWS_PER_TILE, ROWS_PER_TILE)])

    return deg_sc, edge_sc


def _prep_body(dega_ref, degb_ref, x_ref, w_ref, y_ref, dinv_ref):
    deg = dega_ref[:, :1] + degb_ref[:, :1] + 1.0
    dinv = lax.rsqrt(deg)
    xw = jnp.dot(x_ref[...], w_ref[...],
                 preferred_element_type=jnp.float32, precision=_HIGH)
    y_ref[...] = xw * dinv
    dinv_ref[...] = dinv


def _prep_tc(dega, degb, xp, W1):
    return pl.pallas_call(
        _prep_body,
        grid=(8,),
        in_specs=[
            pl.BlockSpec((BR, F), lambda i: (i, 0)),
            pl.BlockSpec((BR, F), lambda i: (i, 0)),
            pl.BlockSpec((BR, F), lambda i: (i, 0)),
            pl.BlockSpec((F, F), lambda i: (0, 0)),
        ],
        out_specs=[
            pl.BlockSpec((BR, F), lambda i: (i, 0)),
            pl.BlockSpec((BR, 1), lambda i: (i, 0)),
        ],
        out_shape=[
            jax.ShapeDtypeStruct((NP, F), jnp.float32),
            jax.ShapeDtypeStruct((NP, 1), jnp.float32),
        ],
    )(dega, degb, xp, W1)


def _mid_body(dinv_ref, agga_ref, aggb_ref, y_ref, b_ref, w_ref, yn_ref):
    dinv = dinv_ref[...]
    h = jnp.maximum(
        dinv * (agga_ref[...] + aggb_ref[...] + y_ref[...]) + b_ref[...], 0.0)
    yn_ref[...] = jnp.dot(h, w_ref[...], preferred_element_type=jnp.float32,
                          precision=_HIGH) * dinv


def _mid_tc(dinv, agga, aggb, y, b, W):
    return pl.pallas_call(
        _mid_body,
        grid=(8,),
        in_specs=[
            pl.BlockSpec((BR, 1), lambda i: (i, 0)),
            pl.BlockSpec((BR, F), lambda i: (i, 0)),
            pl.BlockSpec((BR, F), lambda i: (i, 0)),
            pl.BlockSpec((BR, F), lambda i: (i, 0)),
            pl.BlockSpec((1, F), lambda i: (0, 0)),
            pl.BlockSpec((F, F), lambda i: (0, 0)),
        ],
        out_specs=pl.BlockSpec((BR, F), lambda i: (i, 0)),
        out_shape=jax.ShapeDtypeStruct((NP, F), jnp.float32),
    )(dinv, agga, aggb, y, b, W)


def _final_body(dinv_ref, agga_ref, aggb_ref, y_ref, b_ref, batch_ref,
                lw1_ref, lb1_ref, lw2_ref, lb2_ref, ow_ref, ob_ref,
                out_ref, pooled_scr, cnt_scr):
    i = pl.program_id(0)
    dinv = dinv_ref[...]
    h = jnp.maximum(
        dinv * (agga_ref[...] + aggb_ref[...] + y_ref[...]) + b_ref[...], 0.0)
    brow = jnp.broadcast_to(batch_ref[0], (NUM_GRAPHS, BR))
    giota = lax.broadcasted_iota(jnp.int32, (NUM_GRAPHS, BR), 0)
    oh = (brow == giota).astype(jnp.float32)
    pooled = jnp.dot(oh, h, preferred_element_type=jnp.float32,
                     precision=_HIGH)
    cnt = jnp.sum(oh, axis=1, keepdims=True)

    @pl.when(i == 0)
    def _():
        pooled_scr[...] = pooled
        cnt_scr[...] = cnt

    @pl.when(i > 0)
    def _():
        pooled_scr[...] += pooled
        cnt_scr[...] += cnt

    @pl.when(i == 7)
    def _():
        g = pooled_scr[...] / jnp.maximum(cnt_scr[...], 1.0)
        g = jnp.maximum(
            jnp.dot(g, lw1_ref[...], preferred_element_type=jnp.float32,
                    precision=_HIGH) + lb1_ref[...], 0.0)
        g = jnp.maximum(
            jnp.dot(g, lw2_ref[...], preferred_element_type=jnp.float32,
                    precision=_HIGH) + lb2_ref[...], 0.0)
        out_ref[...] = jnp.dot(g, ow_ref[...],
                               preferred_element_type=jnp.float32,
                               precision=_HIGH) + ob_ref[...]


def _final_tc(dinv, agga, aggb, y, b, batchp, LW1, Lb1, LW2, Lb2, OW, Ob):
    full = lambda shape: pl.BlockSpec(shape, lambda i: tuple(0 for _ in shape))
    return pl.pallas_call(
        _final_body,
        grid=(8,),
        in_specs=[
            pl.BlockSpec((BR, 1), lambda i: (i, 0)),
            pl.BlockSpec((BR, F), lambda i: (i, 0)),
            pl.BlockSpec((BR, F), lambda i: (i, 0)),
            pl.BlockSpec((BR, F), lambda i: (i, 0)),
            full((1, F)),
            pl.BlockSpec((1, 1, BR), lambda i: (i, 0, 0)),
            full((F, F)),
            full((1, F)),
            full((F, 64)),
            full((1, 64)),
            full((64, 10)),
            full((1, 10)),
        ],
        out_specs=pl.BlockSpec((NUM_GRAPHS, 10), lambda i: (0, 0)),
        out_shape=jax.ShapeDtypeStruct((NUM_GRAPHS, 10), jnp.float32),
        scratch_shapes=[
            pltpu.VMEM((NUM_GRAPHS, F), jnp.float32),
            pltpu.VMEM((NUM_GRAPHS, 1), jnp.float32),
        ],
    )(dinv, agga, aggb, y, b, batchp, LW1, Lb1, LW2, Lb2, OW, Ob)


def kernel(x, edge_index, batch, W1, b1, W2, b2, W3, b3,
           LW1, Lb1, LW2, Lb2, OW, Ob):
    src = edge_index[0].astype(jnp.int32)
    dst = edge_index[1].astype(jnp.int32)
    pad_e = EP - N_EDGES
    src_flat = jnp.concatenate([src, jnp.zeros((pad_e,), jnp.int32)])
    dst_flat = jnp.concatenate(
        [dst, N_NODES + jnp.arange(pad_e, dtype=jnp.int32)
         % (NP - N_NODES)])

    def asym(flat):
        ca = 16 * CHA * K
        a0 = flat[:ca].reshape(16, CHA, K)
        a1 = jnp.pad(flat[ca:].reshape(16, CHB, K),
                     ((0, 0), (0, CHA - CHB), (0, 0)))
        return jnp.stack([a0, a1], axis=1).reshape(NW, CHA, K)

    srcp = asym(src_flat)
    dstp = asym(dst_flat)
    dst_deg = dst_flat.reshape(NW, CH, K)
    xp = jnp.zeros((NP, F), jnp.float32).at[:N_NODES].set(x)
    batchp = jnp.concatenate(
        [batch.astype(jnp.int32),
         jnp.full((NP - N_NODES,), 127, jnp.int32)]).reshape(8, 1, BR)
    zeros_f = jnp.zeros((NP, F), jnp.float32)
    ones_f = jnp.ones((K, F), jnp.float32)

    deg_sc, edge_sc = _sc_kernels()
    deg = deg_sc(dst_deg, zeros_f, ones_f)
    y1, dinv = _prep_tc(deg[0], deg[1], xp, W1)
    agg1 = edge_sc(srcp, dstp, y1, zeros_f)
    y2 = _mid_tc(dinv, agg1[0], agg1[1], y1, b1.reshape(1, F), W2)
    agg2 = edge_sc(srcp, dstp, y2, zeros_f)
    y3 = _mid_tc(dinv, agg2[0], agg2[1], y2, b2.reshape(1, F), W3)
    agg3 = edge_sc(srcp, dstp, y3, zeros_f)
    return _final_tc(dinv, agg3[0], agg3[1], y3, b3.reshape(1, F), batchp,
                     LW1, Lb1.reshape(1, F), LW2, Lb2.reshape(1, 64),
                     OW, Ob.reshape(1, 10))
